# baseline (speedup 1.0000x reference)
import functools

import jax
import jax.numpy as jnp
from jax import lax
from jax.experimental import pallas as pl
from jax.experimental.pallas import tpu as pltpu
from jax.experimental.pallas import tpu_sc as plsc

_IMAGE_TOKEN_ID = 1

_N = 16384
_D = 2048
_NC = 2
_NS = 16
_NW = _NC * _NS
_CHUNK = _N // _NW
_L = 16
_K = 8


def _mixer_body(ids_hbm, emb_hbm, src_hbm, out_hbm, ids_v, cidx_v, mask_v, sem):
    wid = lax.axis_index("s") * _NC + lax.axis_index("c")
    base_tok = wid * _CHUNK

    pltpu.sync_copy(ids_hbm, ids_v)

    def pre_body(j, acc):
        v = ids_v[pl.ds(j * _L, _L)]
        return acc + (v == _IMAGE_TOKEN_ID).astype(jnp.int32)

    acc = lax.fori_loop(0, wid * (_CHUNK // _L), pre_body,
                        jnp.zeros((_L,), jnp.int32))
    base = jnp.sum(acc)

    def chunk_body(j, running):
        v = ids_v[pl.ds(base_tok + j * _L, _L)]
        mi = (v == _IMAGE_TOKEN_ID).astype(jnp.int32)
        local = plsc.cumsum(mi)
        cidx_v[pl.ds(j * _L, _L)] = local + (running - 1)
        mask_v[pl.ds(j * _L, _L)] = mi
        return running + jnp.max(local)

    lax.fori_loop(0, _CHUNK // _L, chunk_body, base)

    def dma_grp(g, carry):
        mv = mask_v[pl.ds(g * _L, _L)]
        cv = cidx_v[pl.ds(g * _L, _L)]
        for j in range(_L):
            t = g * _L + j
            pos = base_tok + t
            m = mv[j]
            c = cv[j]

            @pl.when(m == 1)
            def _():
                pltpu.async_copy(src_hbm.at[c], out_hbm.at[pos], sem)

            @pl.when(m != 1)
            def _():
                pltpu.async_copy(emb_hbm.at[pos], out_hbm.at[pos], sem)

            @pl.when(t >= _K)
            def _():
                pltpu.make_async_copy(emb_hbm.at[0], out_hbm.at[0], sem).wait()

        return carry

    lax.fori_loop(0, _CHUNK // _L, dma_grp, 0)

    def drain_body(t, carry):
        pltpu.make_async_copy(emb_hbm.at[0], out_hbm.at[0], sem).wait()
        return carry

    lax.fori_loop(0, _K, drain_body, 0)


@functools.cache
def _mixer():
    return pl.kernel(
        _mixer_body,
        out_type=jax.ShapeDtypeStruct((_N, _D), jnp.float32),
        mesh=plsc.VectorSubcoreMesh(core_axis_name="c", subcore_axis_name="s",
                                    num_cores=_NC, num_subcores=_NS),
        scratch_types=[
            pltpu.VMEM((_N,), jnp.int32),
            pltpu.VMEM((_CHUNK,), jnp.int32),
            pltpu.VMEM((_CHUNK,), jnp.int32),
            pltpu.SemaphoreType.DMA,
        ],
        compiler_params=pltpu.CompilerParams(needs_layout_passes=False),
    )


def kernel(input_ids, inputs_embeds, image_features):
    B, S, D = inputs_embeds.shape
    ids = input_ids.reshape(B * S).astype(jnp.int32)
    emb = inputs_embeds.reshape(B * S, D)
    src = image_features.reshape(-1, D)
    out = _mixer()(ids, emb, src)
    return out.reshape(B, S, D)


# speedup vs baseline: 27.4905x; 27.4905x over previous
import functools

import jax
import jax.numpy as jnp
from jax import lax
from jax.experimental import pallas as pl
from jax.experimental.pallas import tpu as pltpu
from jax.experimental.pallas import tpu_sc as plsc

_IMAGE_TOKEN_ID = 1

_N = 16384
_D = 2048
_NC = 2
_NS = 16
_NW = _NC * _NS
_CHUNK = _N // _NW
_L = 16
_G = 16
_NG = _CHUNK // _G


def _mixer_body(ids_hbm, emb_hbm, src_hbm, out_hbm,
                ids_v, gidx, midx, uidx, buf, gsem, ssem):
    wid = lax.axis_index("s") * _NC + lax.axis_index("c")
    base_tok = wid * _CHUNK
    lanes = lax.iota(jnp.int32, _L)

    pltpu.sync_copy(ids_hbm, ids_v)

    def pre_body(j, acc):
        v = ids_v[pl.ds(j * _L, _L)]
        return acc + (v == _IMAGE_TOKEN_ID).astype(jnp.int32)

    acc = lax.fori_loop(0, wid * (_CHUNK // _L), pre_body,
                        jnp.zeros((_L,), jnp.int32))
    base_m = jnp.sum(acc)

    def scan1(j, carry):
        cnt, lmp, lup = carry
        v = ids_v[pl.ds(base_tok + j * _L, _L)]
        mi = (v == _IMAGE_TOKEN_ID).astype(jnp.int32)
        gpos = base_tok + j * _L + lanes
        lmp = jnp.maximum(lmp, jnp.max(jnp.where(mi == 1, gpos, -1)))
        lup = jnp.maximum(lup, jnp.max(jnp.where(mi == 1, -1, gpos)))
        return cnt + jnp.sum(mi), lmp, lup

    cnt_m, lmp, lup = lax.fori_loop(
        0, _CHUNK // _L, scan1,
        (jnp.int32(0), jnp.int32(-1), jnp.int32(-1)))
    cnt_u = _CHUNK - cnt_m

    def fill(j, _):
        midx[j, :] = jnp.broadcast_to(lmp, (_L,))
        uidx[j, :] = jnp.broadcast_to(lup, (_L,))
        gidx[j, :] = base_m + jnp.minimum(j * _L + lanes, cnt_m - 1)
        return 0

    lax.fori_loop(0, _NG, fill, 0)

    def scan2(j, c):
        v = ids_v[pl.ds(base_tok + j * _L, _L)]
        mi = (v == _IMAGE_TOKEN_ID).astype(jnp.int32)
        csum = plsc.cumsum(mi)
        gpos = base_tok + j * _L + lanes
        lr = jnp.clip(c + csum - 1, 0, _CHUNK - 1)
        ur = jnp.clip(j * _L + lanes - (c + csum), 0, _CHUNK - 1)
        plsc.store_scatter(midx, [lr >> 4, lr & (_G - 1)], gpos,
                           mask=(mi == 1))
        plsc.store_scatter(uidx, [ur >> 4, ur & (_G - 1)], gpos,
                           mask=(mi == 0))
        return c + jnp.max(csum)

    lax.fori_loop(0, _CHUNK // _L, scan2, jnp.int32(0))

    def stream(table, idxg, idxs, trips):
        def body(g, _):
            pltpu.async_copy(table.at[idxg.at[g]], buf, gsem).wait()
            pltpu.async_copy(buf, out_hbm.at[idxs.at[g]], ssem).wait()
            return 0
        lax.fori_loop(0, trips, body, 0)

    stream(src_hbm, gidx, midx, (cnt_m + _G - 1) >> 4)
    stream(emb_hbm, uidx, uidx, (cnt_u + _G - 1) >> 4)


@functools.cache
def _mixer():
    return pl.kernel(
        _mixer_body,
        out_type=jax.ShapeDtypeStruct((_N, _D), jnp.float32),
        mesh=plsc.VectorSubcoreMesh(core_axis_name="c", subcore_axis_name="s",
                                    num_cores=_NC, num_subcores=_NS),
        scratch_types=[
            pltpu.VMEM((_N,), jnp.int32),
            pltpu.VMEM((_NG, _G), jnp.int32),
            pltpu.VMEM((_NG, _G), jnp.int32),
            pltpu.VMEM((_NG, _G), jnp.int32),
            pltpu.VMEM((_G, _D), jnp.float32),
            pltpu.SemaphoreType.DMA,
            pltpu.SemaphoreType.DMA,
        ],
        compiler_params=pltpu.CompilerParams(needs_layout_passes=False),
    )


def kernel(input_ids, inputs_embeds, image_features):
    B, S, D = inputs_embeds.shape
    ids = input_ids.reshape(B * S).astype(jnp.int32)
    emb = inputs_embeds.reshape(B * S, D)
    src = image_features.reshape(-1, D)
    out = _mixer()(ids, emb, src)
    return out.reshape(B, S, D)


# speedup vs baseline: 31.5295x; 1.1469x over previous
import functools

import jax
import jax.numpy as jnp
from jax import lax
from jax.experimental import pallas as pl
from jax.experimental.pallas import tpu as pltpu
from jax.experimental.pallas import tpu_sc as plsc

_IMAGE_TOKEN_ID = 1

_N = 16384
_D = 2048
_NC = 2
_NS = 16
_NW = _NC * _NS
_CHUNK = _N // _NW
_L = 16
_G = 16
_NG = _CHUNK // _G


def _mixer_body(ids_hbm, emb_hbm, src_hbm, out_hbm,
                ids_v, gidx, midx, uidx, buf0, buf1,
                gsem0, gsem1, ssem0, ssem1):
    wid = lax.axis_index("s") * _NC + lax.axis_index("c")
    base_tok = wid * _CHUNK
    lanes = lax.iota(jnp.int32, _L)

    pltpu.sync_copy(ids_hbm, ids_v)

    def pre_body(j, acc):
        v = ids_v[pl.ds(j * _L, _L)]
        return acc + (v == _IMAGE_TOKEN_ID).astype(jnp.int32)

    acc = lax.fori_loop(0, wid * (_CHUNK // _L), pre_body,
                        jnp.zeros((_L,), jnp.int32))
    base_m = jnp.sum(acc)

    def scan1(j, carry):
        cnt, lmp, lup = carry
        v = ids_v[pl.ds(base_tok + j * _L, _L)]
        mi = (v == _IMAGE_TOKEN_ID).astype(jnp.int32)
        gpos = base_tok + j * _L + lanes
        lmp = jnp.maximum(lmp, jnp.max(jnp.where(mi == 1, gpos, -1)))
        lup = jnp.maximum(lup, jnp.max(jnp.where(mi == 1, -1, gpos)))
        return cnt + jnp.sum(mi), lmp, lup

    cnt_m, lmp, lup = lax.fori_loop(
        0, _CHUNK // _L, scan1,
        (jnp.int32(0), jnp.int32(-1), jnp.int32(-1)))
    cnt_u = _CHUNK - cnt_m

    def fill(j, _):
        midx[j, :] = jnp.broadcast_to(lmp, (_L,))
        uidx[j, :] = jnp.broadcast_to(lup, (_L,))
        gidx[j, :] = base_m + jnp.minimum(j * _L + lanes, cnt_m - 1)
        return 0

    lax.fori_loop(0, _NG, fill, 0)

    def scan2(j, c):
        v = ids_v[pl.ds(base_tok + j * _L, _L)]
        mi = (v == _IMAGE_TOKEN_ID).astype(jnp.int32)
        csum = plsc.cumsum(mi)
        gpos = base_tok + j * _L + lanes
        lr = jnp.clip(c + csum - 1, 0, _CHUNK - 1)
        ur = jnp.clip(j * _L + lanes - (c + csum), 0, _CHUNK - 1)
        plsc.store_scatter(midx, [lr >> 4, lr & (_G - 1)], gpos,
                           mask=(mi == 1))
        plsc.store_scatter(uidx, [ur >> 4, ur & (_G - 1)], gpos,
                           mask=(mi == 0))
        return c + jnp.max(csum)

    lax.fori_loop(0, _CHUNK // _L, scan2, jnp.int32(0))

    bufs = (buf0, buf1)
    gsems = (gsem0, gsem1)
    ssems = (ssem0, ssem1)

    def stream(table, idxg, idxs, trips):
        def slot(g, p, pair):
            @pl.when(pair >= 1)
            def _():
                pltpu.make_async_copy(
                    bufs[p], out_hbm.at[pl.ds(0, _G)], ssems[p]).wait()

            pltpu.async_copy(table.at[idxg.at[g]], bufs[p], gsems[p]).wait()
            pltpu.async_copy(bufs[p], out_hbm.at[idxs.at[g]], ssems[p])

        def pair_body(pair, _):
            g0 = 2 * pair
            slot(g0, 0, pair)

            @pl.when(g0 + 1 < trips)
            def _():
                slot(g0 + 1, 1, pair)

            return 0

        lax.fori_loop(0, (trips + 1) >> 1, pair_body, 0)

        @pl.when(trips >= 1)
        def _():
            pltpu.make_async_copy(
                bufs[0], out_hbm.at[pl.ds(0, _G)], ssems[0]).wait()

        @pl.when(trips >= 2)
        def _():
            pltpu.make_async_copy(
                bufs[1], out_hbm.at[pl.ds(0, _G)], ssems[1]).wait()

    stream(src_hbm, gidx, midx, (cnt_m + _G - 1) >> 4)
    stream(emb_hbm, uidx, uidx, (cnt_u + _G - 1) >> 4)


@functools.cache
def _mixer():
    return pl.kernel(
        _mixer_body,
        out_type=jax.ShapeDtypeStruct((_N, _D), jnp.float32),
        mesh=plsc.VectorSubcoreMesh(core_axis_name="c", subcore_axis_name="s",
                                    num_cores=_NC, num_subcores=_NS),
        scratch_types=[
            pltpu.VMEM((_N,), jnp.int32),
            pltpu.VMEM((_NG, _G), jnp.int32),
            pltpu.VMEM((_NG, _G), jnp.int32),
            pltpu.VMEM((_NG, _G), jnp.int32),
            pltpu.VMEM((_G, _D), jnp.float32),
            pltpu.VMEM((_G, _D), jnp.float32),
            pltpu.SemaphoreType.DMA,
            pltpu.SemaphoreType.DMA,
            pltpu.SemaphoreType.DMA,
            pltpu.SemaphoreType.DMA,
        ],
        compiler_params=pltpu.CompilerParams(needs_layout_passes=False),
    )


def kernel(input_ids, inputs_embeds, image_features):
    B, S, D = inputs_embeds.shape
    ids = input_ids.reshape(B * S).astype(jnp.int32)
    emb = inputs_embeds.reshape(B * S, D)
    src = image_features.reshape(-1, D)
    out = _mixer()(ids, emb, src)
    return out.reshape(B, S, D)


# speedup vs baseline: 32.3309x; 1.0254x over previous
import functools

import jax
import jax.numpy as jnp
from jax import lax
from jax.experimental import pallas as pl
from jax.experimental.pallas import tpu as pltpu
from jax.experimental.pallas import tpu_sc as plsc

_IMAGE_TOKEN_ID = 1

_N = 16384
_D = 2048
_NC = 2
_NS = 16
_NW = _NC * _NS
_CHUNK = _N // _NW
_L = 16
_G = 16
_NG = _CHUNK // _G


def _mixer_body(ids_hbm, emb_hbm, src_hbm, out_hbm,
                ids_v, gidx, midx, uidx, buf0, buf1, buf2,
                gsem0, gsem1, gsem2, ssem0, ssem1, ssem2):
    wid = lax.axis_index("s") * _NC + lax.axis_index("c")
    base_tok = wid * _CHUNK
    lanes = lax.iota(jnp.int32, _L)

    pltpu.sync_copy(ids_hbm, ids_v)

    def pre_body(j, acc):
        v = ids_v[pl.ds(j * _L, _L)]
        return acc + (v == _IMAGE_TOKEN_ID).astype(jnp.int32)

    acc = lax.fori_loop(0, wid * (_CHUNK // _L), pre_body,
                        jnp.zeros((_L,), jnp.int32))
    base_m = jnp.sum(acc)

    def scan1(j, carry):
        cnt, lmp, lup = carry
        v = ids_v[pl.ds(base_tok + j * _L, _L)]
        mi = (v == _IMAGE_TOKEN_ID).astype(jnp.int32)
        gpos = base_tok + j * _L + lanes
        lmp = jnp.maximum(lmp, jnp.max(jnp.where(mi == 1, gpos, -1)))
        lup = jnp.maximum(lup, jnp.max(jnp.where(mi == 1, -1, gpos)))
        return cnt + jnp.sum(mi), lmp, lup

    cnt_m, lmp, lup = lax.fori_loop(
        0, _CHUNK // _L, scan1,
        (jnp.int32(0), jnp.int32(-1), jnp.int32(-1)))
    cnt_u = _CHUNK - cnt_m

    def fill(j, _):
        midx[j, :] = jnp.broadcast_to(lmp, (_L,))
        uidx[j, :] = jnp.broadcast_to(lup, (_L,))
        gidx[j, :] = base_m + jnp.minimum(j * _L + lanes, cnt_m - 1)
        return 0

    lax.fori_loop(0, _NG, fill, 0)

    def scan2(j, c):
        v = ids_v[pl.ds(base_tok + j * _L, _L)]
        mi = (v == _IMAGE_TOKEN_ID).astype(jnp.int32)
        csum = plsc.cumsum(mi)
        gpos = base_tok + j * _L + lanes
        lr = jnp.clip(c + csum - 1, 0, _CHUNK - 1)
        ur = jnp.clip(j * _L + lanes - (c + csum), 0, _CHUNK - 1)
        plsc.store_scatter(midx, [lr >> 4, lr & (_G - 1)], gpos,
                           mask=(mi == 1))
        plsc.store_scatter(uidx, [ur >> 4, ur & (_G - 1)], gpos,
                           mask=(mi == 0))
        return c + jnp.max(csum)

    lax.fori_loop(0, _CHUNK // _L, scan2, jnp.int32(0))

    bufs = (buf0, buf1, buf2)
    gsems = (gsem0, gsem1, gsem2)
    ssems = (ssem0, ssem1, ssem2)

    def drain_s(b):
        pltpu.make_async_copy(bufs[b], out_hbm.at[pl.ds(0, _G)],
                              ssems[b]).wait()

    def drain_g(table, b):
        pltpu.make_async_copy(table.at[pl.ds(0, _G)], bufs[b],
                              gsems[b]).wait()

    def stream(table, idxg, idxs, trips):
        def slot(g, b, bp):
            @pl.when(g < trips)
            def _():
                @pl.when(g >= 3)
                def _():
                    drain_s(b)

                pltpu.async_copy(table.at[idxg.at[g]], bufs[b], gsems[b])

                @pl.when(g >= 1)
                def _():
                    drain_g(table, bp)
                    pltpu.async_copy(bufs[bp], out_hbm.at[idxs.at[g - 1]],
                                     ssems[bp])

        def tri(t, _):
            g = 3 * t
            slot(g, 0, 2)
            slot(g + 1, 1, 0)
            slot(g + 2, 2, 1)
            return 0

        lax.fori_loop(0, (trips + 2) // 3, tri, 0)

        last = trips - 1
        for r in range(3):
            @pl.when((trips >= 1) & (last % 3 == r))
            def _():
                drain_g(table, r)
                pltpu.async_copy(bufs[r], out_hbm.at[idxs.at[last]],
                                 ssems[r])

        for r in range(3):
            @pl.when(trips >= r + 1)
            def _():
                drain_s(r)

    stream(src_hbm, gidx, midx, (cnt_m + _G - 1) >> 4)
    stream(emb_hbm, uidx, uidx, (cnt_u + _G - 1) >> 4)


@functools.cache
def _mixer():
    return pl.kernel(
        _mixer_body,
        out_type=jax.ShapeDtypeStruct((_N, _D), jnp.float32),
        mesh=plsc.VectorSubcoreMesh(core_axis_name="c", subcore_axis_name="s",
                                    num_cores=_NC, num_subcores=_NS),
        scratch_types=[
            pltpu.VMEM((_N,), jnp.int32),
            pltpu.VMEM((_NG, _G), jnp.int32),
            pltpu.VMEM((_NG, _G), jnp.int32),
            pltpu.VMEM((_NG, _G), jnp.int32),
            pltpu.VMEM((_G, _D), jnp.float32),
            pltpu.VMEM((_G, _D), jnp.float32),
            pltpu.VMEM((_G, _D), jnp.float32),
            pltpu.SemaphoreType.DMA,
            pltpu.SemaphoreType.DMA,
            pltpu.SemaphoreType.DMA,
            pltpu.SemaphoreType.DMA,
            pltpu.SemaphoreType.DMA,
            pltpu.SemaphoreType.DMA,
        ],
        compiler_params=pltpu.CompilerParams(needs_layout_passes=False),
    )


def kernel(input_ids, inputs_embeds, image_features):
    B, S, D = inputs_embeds.shape
    ids = input_ids.reshape(B * S).astype(jnp.int32)
    emb = inputs_embeds.reshape(B * S, D)
    src = image_features.reshape(-1, D)
    out = _mixer()(ids, emb, src)
    return out.reshape(B, S, D)


# speedup vs baseline: 32.5596x; 1.0071x over previous
import functools

import jax
import jax.numpy as jnp
from jax import lax
from jax.experimental import pallas as pl
from jax.experimental.pallas import tpu as pltpu
from jax.experimental.pallas import tpu_sc as plsc

_IMAGE_TOKEN_ID = 1

_N = 16384
_D = 2048
_NC = 2
_NS = 16
_NW = _NC * _NS
_CHUNK = _N // _NW
_L = 16
_G = 16
_NG = _CHUNK // _G


def _mixer_body(ids_hbm, emb_hbm, src_hbm, out_hbm,
                ids_v, gidx, sidx, buf0, buf1, buf2,
                gsem0, gsem1, gsem2, ssem0, ssem1, ssem2):
    wid = lax.axis_index("s") * _NC + lax.axis_index("c")
    base_tok = wid * _CHUNK
    lanes = lax.iota(jnp.int32, _L)

    pltpu.sync_copy(ids_hbm, ids_v)

    def pre_body(j, acc):
        v = ids_v[pl.ds(j * _L, _L)]
        return acc + (v == _IMAGE_TOKEN_ID).astype(jnp.int32)

    acc = lax.fori_loop(0, wid * (_CHUNK // _L), pre_body,
                        jnp.zeros((_L,), jnp.int32))
    base_m = jnp.sum(acc)

    def scan1(j, carry):
        cnt, lmp, lup = carry
        v = ids_v[pl.ds(base_tok + j * _L, _L)]
        mi = (v == _IMAGE_TOKEN_ID).astype(jnp.int32)
        gpos = base_tok + j * _L + lanes
        lmp = jnp.maximum(lmp, jnp.max(jnp.where(mi == 1, gpos, -1)))
        lup = jnp.maximum(lup, jnp.max(jnp.where(mi == 1, -1, gpos)))
        return cnt + jnp.sum(mi), lmp, lup

    cnt_m, lmp, lup = lax.fori_loop(
        0, _CHUNK // _L, scan1,
        (jnp.int32(0), jnp.int32(-1), jnp.int32(-1)))
    cnt_u = _CHUNK - cnt_m
    tm = (cnt_m + _G - 1) >> 4
    tu = (cnt_u + _G - 1) >> 4

    def fill(j, _):
        sidx[j, :] = jnp.broadcast_to(jnp.where(j < tm, lmp, lup), (_L,))
        @pl.when(j < _NG)
        def _():
            gidx[j, :] = base_m + jnp.minimum(j * _L + lanes, cnt_m - 1)
        return 0

    lax.fori_loop(0, _NG + 1, fill, 0)

    def scan2(j, c):
        v = ids_v[pl.ds(base_tok + j * _L, _L)]
        mi = (v == _IMAGE_TOKEN_ID).astype(jnp.int32)
        csum = plsc.cumsum(mi)
        gpos = base_tok + j * _L + lanes
        lr = jnp.clip(c + csum - 1, 0, jnp.maximum(cnt_m - 1, 0))
        ur = jnp.clip(j * _L + lanes - (c + csum), 0,
                      jnp.maximum(cnt_u - 1, 0))
        plsc.store_scatter(sidx, [lr >> 4, lr & (_G - 1)], gpos,
                           mask=(mi == 1))
        plsc.store_scatter(sidx, [tm + (ur >> 4), ur & (_G - 1)], gpos,
                           mask=(mi == 0))
        return c + jnp.max(csum)

    lax.fori_loop(0, _CHUNK // _L, scan2, jnp.int32(0))

    bufs = (buf0, buf1, buf2)
    gsems = (gsem0, gsem1, gsem2)
    ssems = (ssem0, ssem1, ssem2)
    trips = tm + tu

    def drain_s(b):
        pltpu.make_async_copy(bufs[b], out_hbm.at[pl.ds(0, _G)],
                              ssems[b]).wait()

    def drain_g(b):
        pltpu.make_async_copy(src_hbm.at[pl.ds(0, _G)], bufs[b],
                              gsems[b]).wait()

    def gather(g, b):
        @pl.when(g < tm)
        def _():
            pltpu.async_copy(src_hbm.at[gidx.at[g]], bufs[b], gsems[b])

        @pl.when(g >= tm)
        def _():
            pltpu.async_copy(emb_hbm.at[sidx.at[g]], bufs[b], gsems[b])

    def slot(g, b, bp):
        @pl.when(g < trips)
        def _():
            @pl.when(g >= 3)
            def _():
                drain_s(b)

            gather(g, b)

            @pl.when(g >= 1)
            def _():
                drain_g(bp)
                pltpu.async_copy(bufs[bp], out_hbm.at[sidx.at[g - 1]],
                                 ssems[bp])

    def tri(t, _):
        g = 3 * t
        slot(g, 0, 2)
        slot(g + 1, 1, 0)
        slot(g + 2, 2, 1)
        return 0

    lax.fori_loop(0, (trips + 2) // 3, tri, 0)

    last = trips - 1
    for r in range(3):
        @pl.when((trips >= 1) & (last % 3 == r))
        def _():
            drain_g(r)
            pltpu.async_copy(bufs[r], out_hbm.at[sidx.at[last]],
                             ssems[r])

    for r in range(3):
        @pl.when(trips >= r + 1)
        def _():
            drain_s(r)


@functools.cache
def _mixer():
    return pl.kernel(
        _mixer_body,
        out_type=jax.ShapeDtypeStruct((_N, _D), jnp.float32),
        mesh=plsc.VectorSubcoreMesh(core_axis_name="c", subcore_axis_name="s",
                                    num_cores=_NC, num_subcores=_NS),
        scratch_types=[
            pltpu.VMEM((_N,), jnp.int32),
            pltpu.VMEM((_NG, _G), jnp.int32),
            pltpu.VMEM((_NG + 1, _G), jnp.int32),
            pltpu.VMEM((_G, _D), jnp.float32),
            pltpu.VMEM((_G, _D), jnp.float32),
            pltpu.VMEM((_G, _D), jnp.float32),
            pltpu.SemaphoreType.DMA,
            pltpu.SemaphoreType.DMA,
            pltpu.SemaphoreType.DMA,
            pltpu.SemaphoreType.DMA,
            pltpu.SemaphoreType.DMA,
            pltpu.SemaphoreType.DMA,
        ],
        compiler_params=pltpu.CompilerParams(needs_layout_passes=False),
    )


def kernel(input_ids, inputs_embeds, image_features):
    B, S, D = inputs_embeds.shape
    ids = input_ids.reshape(B * S).astype(jnp.int32)
    emb = inputs_embeds.reshape(B * S, D)
    src = image_features.reshape(-1, D)
    out = _mixer()(ids, emb, src)
    return out.reshape(B, S, D)


# speedup vs baseline: 33.6925x; 1.0348x over previous
import functools

import jax
import jax.numpy as jnp
from jax import lax
from jax.experimental import pallas as pl
from jax.experimental.pallas import tpu as pltpu
from jax.experimental.pallas import tpu_sc as plsc

_IMAGE_TOKEN_ID = 1

_N = 16384
_D = 2048
_NC = 2
_NS = 16
_NW = _NC * _NS
_CHUNK = _N // _NW
_L = 16
_GS = 3
_G = 1 << _GS
_NG = _CHUNK // _G
_K = 6


def _mixer_body(ids_hbm, emb_hbm, src_hbm, out_hbm, ids_v, gidx, sidx,
                *rest):
    bufs = rest[:_K]
    gsems = rest[_K:2 * _K]
    ssems = rest[2 * _K:]
    wid = lax.axis_index("s") * _NC + lax.axis_index("c")
    base_tok = wid * _CHUNK
    lanes = lax.iota(jnp.int32, _L)

    pltpu.sync_copy(ids_hbm, ids_v)

    def pre_body(j, acc):
        v = ids_v[pl.ds(j * _L, _L)]
        return acc + (v == _IMAGE_TOKEN_ID).astype(jnp.int32)

    acc = lax.fori_loop(0, wid * (_CHUNK // _L), pre_body,
                        jnp.zeros((_L,), jnp.int32))
    base_m = jnp.sum(acc)

    def scan1(j, carry):
        cnt, lmp, lup = carry
        v = ids_v[pl.ds(base_tok + j * _L, _L)]
        mi = (v == _IMAGE_TOKEN_ID).astype(jnp.int32)
        gpos = base_tok + j * _L + lanes
        lmp = jnp.maximum(lmp, jnp.max(jnp.where(mi == 1, gpos, -1)))
        lup = jnp.maximum(lup, jnp.max(jnp.where(mi == 1, -1, gpos)))
        return cnt + jnp.sum(mi), lmp, lup

    cnt_m, lmp, lup = lax.fori_loop(
        0, _CHUNK // _L, scan1,
        (jnp.int32(0), jnp.int32(-1), jnp.int32(-1)))
    cnt_u = _CHUNK - cnt_m
    tm = (cnt_m + _G - 1) >> _GS
    tu = (cnt_u + _G - 1) >> _GS

    def fill(j, _):
        pos = j * _L + lanes
        sidx[pl.ds(j * _L, _L)] = jnp.where(pos < tm * _G, lmp, lup)
        @pl.when(j * _L < _CHUNK)
        def _():
            gidx[pl.ds(j * _L, _L)] = base_m + jnp.minimum(pos, cnt_m - 1)
        return 0

    lax.fori_loop(0, (_CHUNK + _L) // _L, fill, 0)

    def scan2(j, c):
        v = ids_v[pl.ds(base_tok + j * _L, _L)]
        mi = (v == _IMAGE_TOKEN_ID).astype(jnp.int32)
        csum = plsc.cumsum(mi)
        gpos = base_tok + j * _L + lanes
        lr = jnp.clip(c + csum - 1, 0, jnp.maximum(cnt_m - 1, 0))
        ur = jnp.clip(j * _L + lanes - (c + csum), 0,
                      jnp.maximum(cnt_u - 1, 0))
        plsc.store_scatter(sidx, [lr], gpos, mask=(mi == 1))
        plsc.store_scatter(sidx, [tm * _G + ur], gpos, mask=(mi == 0))
        return c + jnp.max(csum)

    lax.fori_loop(0, _CHUNK // _L, scan2, jnp.int32(0))

    trips = tm + tu

    def drain_s(b):
        pltpu.make_async_copy(bufs[b], out_hbm.at[pl.ds(0, _G)],
                              ssems[b]).wait()

    def drain_g(b):
        pltpu.make_async_copy(src_hbm.at[pl.ds(0, _G)], bufs[b],
                              gsems[b]).wait()

    def gather(g, b):
        @pl.when(g < tm)
        def _():
            pltpu.async_copy(src_hbm.at[gidx.at[pl.ds(g * _G, _G)]],
                             bufs[b], gsems[b])

        @pl.when(g >= tm)
        def _():
            pltpu.async_copy(emb_hbm.at[sidx.at[pl.ds(g * _G, _G)]],
                             bufs[b], gsems[b])

    def slot(g, b, bp):
        @pl.when(g < trips)
        def _():
            @pl.when(g >= _K)
            def _():
                drain_s(b)

            gather(g, b)

            @pl.when(g >= 1)
            def _():
                drain_g(bp)
                pltpu.async_copy(
                    bufs[bp], out_hbm.at[sidx.at[pl.ds((g - 1) * _G, _G)]],
                    ssems[bp])

    def kslots(t, _):
        g = t * _K
        for i in range(_K):
            slot(g + i, i, (i - 1) % _K)
        return 0

    lax.fori_loop(0, (trips + _K - 1) // _K, kslots, 0)

    last = trips - 1
    for r in range(_K):
        @pl.when((trips >= 1) & (last % _K == r))
        def _():
            drain_g(r)
            pltpu.async_copy(
                bufs[r], out_hbm.at[sidx.at[pl.ds(last * _G, _G)]],
                ssems[r])

    for r in range(_K):
        @pl.when(trips >= r + 1)
        def _():
            drain_s(r)


@functools.cache
def _mixer():
    return pl.kernel(
        _mixer_body,
        out_type=jax.ShapeDtypeStruct((_N, _D), jnp.float32),
        mesh=plsc.VectorSubcoreMesh(core_axis_name="c", subcore_axis_name="s",
                                    num_cores=_NC, num_subcores=_NS),
        scratch_types=[
            pltpu.VMEM((_N,), jnp.int32),
            pltpu.VMEM((_CHUNK,), jnp.int32),
            pltpu.VMEM((_CHUNK + _L,), jnp.int32),
        ] + [pltpu.VMEM((_G, _D), jnp.float32)] * _K
          + [pltpu.SemaphoreType.DMA] * (2 * _K),
        compiler_params=pltpu.CompilerParams(needs_layout_passes=False),
    )


def kernel(input_ids, inputs_embeds, image_features):
    B, S, D = inputs_embeds.shape
    ids = input_ids.reshape(B * S).astype(jnp.int32)
    emb = inputs_embeds.reshape(B * S, D)
    src = image_features.reshape(B * S, D)
    out = _mixer()(ids, emb, src)
    return out.reshape(B, S, D)
